# baseline (speedup 1.0000x reference)
import functools

import jax
import jax.numpy as jnp
from jax import lax
from jax.experimental import pallas as pl
from jax.experimental.pallas import tpu as pltpu
from jax.experimental.pallas import tpu_sc as plsc

_NG = 4
_D = 32
_E = 8
_OC = 16
_B = 64
_L = 4096
_LO = _L - 2
_GD = _NG * _OC
_KG = _D * 5
_NL = 16

_BB = 4


def _sc_gate_body(gxt_hbm, wgb_hbm, gates_hbm, gx_v, wg_v, gblk_v):
    c = lax.axis_index("c")
    s = lax.axis_index("s")
    g = 2 * c + s // 4
    chunk = s % 4
    zeros = jnp.zeros((_NL,), jnp.float32)

    @pl.when(s < 8)
    def _():
        pltpu.sync_copy(gxt_hbm.at[g, chunk], gx_v)
        pltpu.sync_copy(wgb_hbm.at[g], wg_v)

        def body(k, accs):
            xv = gx_v[pl.ds(k * _NL, _NL)]
            return tuple(
                accs[e] + wg_v[pl.ds((e * _KG + k) * _NL, _NL)] * xv
                for e in range(_E))

        logits = lax.fori_loop(
            0, _KG, body, tuple(zeros for _ in range(_E)))
        m = logits[0]
        for e in range(1, _E):
            m = jnp.maximum(m, logits[e])
        p = [jnp.exp(v - m) for v in logits]
        tot = p[0]
        for e in range(1, _E):
            tot = tot + p[e]
        p = [v / tot for v in p]
        first = p[0]
        i1 = jnp.zeros((_NL,), jnp.int32)
        second = jnp.full((_NL,), -1.0, jnp.float32)
        i2 = jnp.zeros((_NL,), jnp.int32)
        for e in range(1, _E):
            c1 = p[e] > first
            c2 = p[e] > second
            second_n = jnp.where(c1, first, jnp.where(c2, p[e], second))
            i2_n = jnp.where(c1, i1, jnp.where(c2, e, i2))
            first = jnp.where(c1, p[e], first)
            i1 = jnp.where(c1, e, i1)
            second = second_n
            i2 = i2_n
        denom = first + second + 1e-6
        g1 = first / denom
        g2 = second / denom
        for e in range(_E):
            gate_e = jnp.where(i1 == e, g1, jnp.where(i2 == e, g2, 0.0))
            gblk_v[e] = gate_e
        pltpu.sync_copy(gblk_v, gates_hbm.at[g, chunk])


def _cv2(v):
    mean = jnp.sum(v, axis=1, keepdims=True) / _E
    var = jnp.sum((v - mean) ** 2, axis=1, keepdims=True) / (_E - 1)
    return var / (mean * mean + 1e-10)


def _main_body(x_ref, w1_ref, b1_ref, w2blk_ref, b2f_ref, gsel_ref, g_ref,
               gfull_ref, out_ref, loss_ref):
    @pl.when(pl.program_id(0) == 0)
    def _():
        loss = jnp.zeros((1, 1), jnp.float32)
        for g in range(_NG):
            gg = gfull_ref[:, g, :]
            imp = jnp.sum(gg, axis=0, keepdims=True)
            load = jnp.sum((gg > 0.0).astype(jnp.float32), axis=0,
                           keepdims=True)
            loss = loss + (_cv2(imp) + _cv2(load)) * 0.01
        loss_ref[...] = loss
    for bb in range(_BB):
        x = x_ref[bb].astype(jnp.bfloat16)
        w1 = w1_ref[...].astype(jnp.bfloat16)
        y0 = jnp.dot(w1[0], x, preferred_element_type=jnp.float32)
        y1 = jnp.dot(w1[1], x, preferred_element_type=jnp.float32)
        y2 = jnp.dot(w1[2], x, preferred_element_type=jnp.float32)
        acc = (y0 + pltpu.roll(y1, _L - 1, 1) + pltpu.roll(y2, _L - 2, 1)
               + b1_ref[...])
        h = jnp.tanh(acc)
        gb = g_ref[bb]
        ge = jnp.dot(gsel_ref[...], gb, preferred_element_type=jnp.float32)
        w2bd = jnp.zeros((_GD, _GD), jnp.float32)
        for e in range(_E):
            w2bd = w2bd + w2blk_ref[e] * ge[:, e:e + 1]
        b2e = jnp.sum(b2f_ref[...] * ge, axis=1, keepdims=True)
        out = jnp.dot(w2bd, h, preferred_element_type=jnp.float32) + b2e
        out_ref[bb] = out[:, 0:_LO]


@functools.partial(jax.jit, static_argnames=("interpret",))
def _run(x, Wg, W1, b1, W2, b2, interpret=False):
    f32 = jnp.float32
    xs = jax.lax.slice_in_dim(x, _L - 6, _L - 1, axis=2)
    gxt = (xs.reshape(_B, _NG, _KG).transpose(1, 2, 0)
           .reshape(_NG, _KG, 4, _NL).transpose(0, 2, 1, 3)
           .reshape(_NG, 4, _KG * _NL))
    gxt = jax.lax.reduce_precision(gxt, exponent_bits=8, mantissa_bits=7)
    wgb = jnp.broadcast_to(
        Wg.transpose(0, 2, 1)[:, :, :, None],
        (_NG, _E, _KG, _NL)).reshape(_NG, _E * _KG * _NL)
    wgb = jax.lax.reduce_precision(wgb, exponent_bits=8, mantissa_bits=7)

    mesh = plsc.VectorSubcoreMesh(core_axis_name="c", subcore_axis_name="s")
    sc_gate = functools.partial(
        pl.kernel,
        out_type=jax.ShapeDtypeStruct((_NG, 4, _E, _NL), f32),
        mesh=mesh,
        scratch_types=[
            pltpu.VMEM((_KG * _NL,), f32),
            pltpu.VMEM((_E * _KG * _NL,), f32),
            pltpu.VMEM((_E, _NL), f32),
        ],
    )(_sc_gate_body)
    gates_sc = sc_gate(gxt, wgb)
    gates_t = gates_sc.transpose(1, 3, 0, 2).reshape(_B, _NG, _E)

    w1bd = jnp.zeros((3, _GD, _NG * _D), f32)
    for g in range(_NG):
        w1bd = jax.lax.dynamic_update_slice(
            w1bd, W1[g].transpose(2, 0, 1), (0, g * _OC, g * _D))
    b1f = b1.reshape(_GD, 1)
    w2r = W2[:, :, :, 0].reshape(_NG, _OC, _E, _OC)
    w2blk = jnp.zeros((_E, _GD, _GD), f32)
    for g in range(_NG):
        w2blk = jax.lax.dynamic_update_slice(
            w2blk, w2r[g].transpose(1, 0, 2), (0, g * _OC, g * _OC))
    b2f = b2.reshape(_NG, _OC, _E).reshape(_GD, _E)
    gsel = jnp.repeat(jnp.eye(_NG, dtype=f32), _OC, axis=0)

    out, loss = pl.pallas_call(
        _main_body,
        grid=(_B // _BB,),
        in_specs=[
            pl.BlockSpec((_BB, _NG * _D, _L), lambda b: (b, 0, 0)),
            pl.BlockSpec((3, _GD, _NG * _D), lambda b: (0, 0, 0)),
            pl.BlockSpec((_GD, 1), lambda b: (0, 0)),
            pl.BlockSpec((_E, _GD, _GD), lambda b: (0, 0, 0)),
            pl.BlockSpec((_GD, _E), lambda b: (0, 0)),
            pl.BlockSpec((_GD, _NG), lambda b: (0, 0)),
            pl.BlockSpec((_BB, _NG, _E), lambda b: (b, 0, 0)),
            pl.BlockSpec((_B, _NG, _E), lambda b: (0, 0, 0)),
        ],
        out_specs=(
            pl.BlockSpec((_BB, _GD, _LO), lambda b: (b, 0, 0)),
            pl.BlockSpec((1, 1), lambda b: (0, 0)),
        ),
        out_shape=(
            jax.ShapeDtypeStruct((_B, _GD, _LO), f32),
            jax.ShapeDtypeStruct((1, 1), f32),
        ),
        interpret=interpret,
    )(x, w1bd, b1f, w2blk, b2f, gsel, gates_t, gates_t)

    combine = out.reshape(_B, _NG, _OC, _LO)
    gates_all = gates_t.transpose(0, 2, 1)
    return combine, loss[0, 0], gates_all


def kernel(x, Wg, W1, b1, W2, b2):
    return _run(x, Wg, W1, b1, W2, b2)
